# baseline (speedup 1.0000x reference)
import functools

import jax
import jax.numpy as jnp
from jax import lax
from jax.experimental import pallas as pl
from jax.experimental.pallas import tpu as pltpu
from jax.experimental.pallas import tpu_sc as plsc

N = 10000
E = 320000
H = 128
STEPS = 3

NC = 2
NS = 16
NW = NC * NS
EPW = E // NW
C = 80
NCHUNK = EPW // C
NPAD = 10240
NPT = NPAD // NS


def _mlp2_body(act_last, x_ref, w0_ref, b0_ref, w1_ref, b1_ref, o_ref):
    h = jnp.maximum(
        jnp.dot(x_ref[...], w0_ref[...], preferred_element_type=jnp.float32)
        + b0_ref[...], 0.0)
    y = jnp.dot(h, w1_ref[...], preferred_element_type=jnp.float32) + b1_ref[...]
    o_ref[...] = jnp.maximum(y, 0.0) if act_last else y


def _mlp2(x, w0, b0, w1, b1, act_last, blk):
    n, d = x.shape
    h = w0.shape[1]
    return pl.pallas_call(
        functools.partial(_mlp2_body, act_last),
        grid=(n // blk,),
        in_specs=[
            pl.BlockSpec((blk, d), lambda i: (i, 0)),
            pl.BlockSpec((d, h), lambda i: (0, 0)),
            pl.BlockSpec((1, h), lambda i: (0, 0)),
            pl.BlockSpec((h, h), lambda i: (0, 0)),
            pl.BlockSpec((1, h), lambda i: (0, 0)),
        ],
        out_specs=pl.BlockSpec((blk, h), lambda i: (i, 0)),
        out_shape=jax.ShapeDtypeStruct((n, h), jnp.float32),
    )(x, w0, b0.reshape(1, h), w1, b1.reshape(1, h))


def _vp_body(v_ref, wd_ref, ws_ref, pd_ref, ps_ref):
    v = v_ref[...]
    pd_ref[...] = jnp.dot(v, wd_ref[...], preferred_element_type=jnp.float32)
    ps_ref[...] = jnp.dot(v, ws_ref[...], preferred_element_type=jnp.float32)


def _vp(v, wd, ws, blk=1000):
    return pl.pallas_call(
        _vp_body,
        grid=(N // blk,),
        in_specs=[
            pl.BlockSpec((blk, H), lambda i: (i, 0)),
            pl.BlockSpec((H, H), lambda i: (0, 0)),
            pl.BlockSpec((H, H), lambda i: (0, 0)),
        ],
        out_specs=[
            pl.BlockSpec((blk, H), lambda i: (i, 0)),
            pl.BlockSpec((blk, H), lambda i: (i, 0)),
        ],
        out_shape=[
            jax.ShapeDtypeStruct((N, H), jnp.float32),
            jax.ShapeDtypeStruct((N, H), jnp.float32),
        ],
    )(v, wd, ws)


def _edge_body(with_residual, e_ref, g_ref, w0_ref, b0_ref, w1_ref, b1_ref,
               en_ref, eo_ref=None):
    e = e_ref[...]
    g32 = g_ref[...]
    ga = lax.bitcast_convert_type(g32 << jnp.int32(16), jnp.float32)
    gb2 = lax.bitcast_convert_type(g32 & jnp.int32(-65536), jnp.float32)
    g = jnp.concatenate([ga, gb2], axis=-1)
    h = jnp.maximum(
        jnp.dot(e, w0_ref[...], preferred_element_type=jnp.float32)
        + g + b0_ref[...], 0.0)
    en = jnp.maximum(
        jnp.dot(h, w1_ref[...], preferred_element_type=jnp.float32)
        + b1_ref[...], 0.0)
    en_ref[...] = en
    if with_residual:
        eo_ref[...] = en + e


def _edge_mlp(e, g, w0e, b0, w1, b1, with_residual, blk=2000):
    n_out = 2 if with_residual else 1
    out = pl.pallas_call(
        functools.partial(_edge_body, with_residual),
        grid=(E // blk,),
        in_specs=[
            pl.BlockSpec((blk, H), lambda i: (i, 0)),
            pl.BlockSpec((blk, H // 2), lambda i: (i, 0)),
            pl.BlockSpec((H, H), lambda i: (0, 0)),
            pl.BlockSpec((1, H), lambda i: (0, 0)),
            pl.BlockSpec((H, H), lambda i: (0, 0)),
            pl.BlockSpec((1, H), lambda i: (0, 0)),
        ],
        out_specs=[pl.BlockSpec((blk, H), lambda i: (i, 0))] * n_out,
        out_shape=[jax.ShapeDtypeStruct((E, H), jnp.float32)] * n_out,
    )(e, g, w0e, b0.reshape(1, H), w1, b1.reshape(1, H))
    return out if with_residual else (out[0], None)


def _node_body(pe_ref, pc_ref, v_ref, w0a_ref, w0v_ref, b0_ref, w1_ref, b1_ref,
               vo_ref):
    s_e = pe_ref[0, :, :] + pe_ref[1, :, :]
    cnt = pc_ref[0, :, 0:1] + pc_ref[1, :, 0:1]
    agg = s_e * (1.0 / jnp.maximum(cnt, 1.0))
    v = v_ref[...]
    h = jnp.maximum(
        jnp.dot(agg, w0a_ref[...], preferred_element_type=jnp.float32)
        + jnp.dot(v, w0v_ref[...], preferred_element_type=jnp.float32)
        + b0_ref[...], 0.0)
    vn = jnp.maximum(
        jnp.dot(h, w1_ref[...], preferred_element_type=jnp.float32)
        + b1_ref[...], 0.0)
    vo_ref[...] = vn + v


def _node_mlp(parts_e, parts_c, v, w0a, w0v, b0, w1, b1, blk=1000):
    return pl.pallas_call(
        _node_body,
        grid=(N // blk,),
        in_specs=[
            pl.BlockSpec((NC, blk, H), lambda i: (0, i, 0)),
            pl.BlockSpec((NC, blk, H), lambda i: (0, i, 0)),
            pl.BlockSpec((blk, H), lambda i: (i, 0)),
            pl.BlockSpec((H, H), lambda i: (0, 0)),
            pl.BlockSpec((H, H), lambda i: (0, 0)),
            pl.BlockSpec((1, H), lambda i: (0, 0)),
            pl.BlockSpec((H, H), lambda i: (0, 0)),
            pl.BlockSpec((1, H), lambda i: (0, 0)),
        ],
        out_specs=pl.BlockSpec((blk, H), lambda i: (i, 0)),
        out_shape=jax.ShapeDtypeStruct((N, H), jnp.float32),
    )(parts_e, parts_c, v, w0a, w0v, b0.reshape(1, H), w1, b1.reshape(1, H))


def _sc_gather_add(pd, ps, dstv, srcv):
    mesh = plsc.VectorSubcoreMesh(core_axis_name="c", subcore_axis_name="s")

    @functools.partial(
        pl.kernel,
        mesh=mesh,
        out_type=jax.ShapeDtypeStruct((E, H // 2), jnp.int32),
        scratch_types=[
            pltpu.VMEM((C,), jnp.int32),
            pltpu.VMEM((C,), jnp.int32),
            pltpu.VMEM((C,), jnp.int32),
            pltpu.VMEM((C,), jnp.int32),
            pltpu.VMEM((C, H), jnp.float32),
            pltpu.VMEM((C, H), jnp.float32),
            pltpu.VMEM((C, H), jnp.float32),
            pltpu.VMEM((C, H), jnp.float32),
            pltpu.VMEM((C, H // 2), jnp.int32),
            pltpu.VMEM((C, H // 2), jnp.int32),
            pltpu.SemaphoreType.DMA,
            pltpu.SemaphoreType.DMA,
            pltpu.SemaphoreType.DMA,
            pltpu.SemaphoreType.DMA,
            pltpu.SemaphoreType.DMA,
            pltpu.SemaphoreType.DMA,
            pltpu.SemaphoreType.DMA,
            pltpu.SemaphoreType.DMA,
        ],
    )
    def k(pd_hbm, ps_hbm, dst_hbm, src_hbm, g_hbm,
          idxd0, idxs0, idxd1, idxs1, rd0, rs0, rd1, rs1, gb0, gb1,
          semd0, sems0, semd1, sems1, semi0, semi1, semo0, semo1):
        wid = lax.axis_index("s") * NC + lax.axis_index("c")
        base0 = wid * EPW
        bufs = ((idxd0, idxs0, rd0, rs0, gb0, semd0, sems0, semi0, semo0),
                (idxd1, idxs1, rd1, rs1, gb1, semd1, sems1, semi1, semo1))

        def idx_start(i, b):
            idxd, idxs = bufs[b][0], bufs[b][1]
            semi = bufs[b][7]
            base = base0 + i * C
            pltpu.async_copy(dst_hbm.at[pl.ds(base, C)], idxd, semi)
            pltpu.async_copy(src_hbm.at[pl.ds(base, C)], idxs, semi)

        def gather_start(b, first):
            idxd, idxs, rd, rs, gb, semd, sems_, semi, semo = bufs[b]
            pltpu.make_async_copy(dst_hbm.at[pl.ds(0, C)], idxd, semi).wait()
            pltpu.make_async_copy(src_hbm.at[pl.ds(0, C)], idxs, semi).wait()
            if not first:
                pltpu.make_async_copy(gb, g_hbm.at[pl.ds(0, C)], semo).wait()
            pltpu.async_copy(pd_hbm.at[idxd], rd, semd)
            pltpu.async_copy(ps_hbm.at[idxs], rs, sems_)

        def finish(i, b):
            idxd, idxs, rd, rs, gb, semd, sems_, _, semo = bufs[b]
            pltpu.make_async_copy(pd_hbm.at[idxd], rd, semd).wait()
            pltpu.make_async_copy(ps_hbm.at[idxs], rs, sems_).wait()

            def row(j, c2):
                for m in range(4):
                    sa = pl.ds(m * 16, 16)
                    sb = pl.ds(m * 16 + 64, 16)
                    au = lax.bitcast_convert_type(rd[j, sa] + rs[j, sa],
                                                  jnp.uint32)
                    bu = lax.bitcast_convert_type(rd[j, sb] + rs[j, sb],
                                                  jnp.uint32)
                    ar = (au + jnp.uint32(0x7FFF)
                          + ((au >> jnp.uint32(16)) & jnp.uint32(1)))
                    br = (bu + jnp.uint32(0x7FFF)
                          + ((bu >> jnp.uint32(16)) & jnp.uint32(1)))
                    word = ((ar >> jnp.uint32(16))
                            | (br & jnp.uint32(0xFFFF0000)))
                    gb[j, pl.ds(m * 16, 16)] = lax.bitcast_convert_type(
                        word, jnp.int32)
                return c2

            lax.fori_loop(0, C, row, 0)
            pltpu.async_copy(gb, g_hbm.at[pl.ds(base0 + i * C, C)], semo)

        idx_start(0, 0)
        gather_start(0, True)
        idx_start(1, 1)
        gather_start(1, True)

        def pair(p, carry):
            finish(2 * p, 0)
            idx_start(2 * p + 2, 0)
            gather_start(0, False)
            finish(2 * p + 1, 1)

            @pl.when(2 * p + 3 < NCHUNK)
            def _():
                idx_start(2 * p + 3, 1)
                gather_start(1, False)

            return carry

        lax.fori_loop(0, (NCHUNK - 1) // 2, pair, 0)
        finish(NCHUNK - 1, 0)
        pltpu.make_async_copy(gb0, g_hbm.at[pl.ds(0, C)], semo0).wait()
        pltpu.make_async_copy(gb1, g_hbm.at[pl.ds(0, C)], semo1).wait()

    return k(pd, ps, dstv, srcv)


def _sc_scatter_add(e_new, dstv):
    mesh = plsc.VectorSubcoreMesh(core_axis_name="c", subcore_axis_name="s")

    @functools.partial(
        pl.kernel,
        mesh=mesh,
        out_type=jax.ShapeDtypeStruct((NC, NPAD, H), jnp.float32),
        scratch_types=[
            pltpu.VMEM((C,), jnp.int32),
            pltpu.VMEM((C, H), jnp.float32),
            pltpu.VMEM((C,), jnp.int32),
            pltpu.VMEM((C, H), jnp.float32),
            pltpu.VMEM((160, H), jnp.float32),
            pltpu.VMEM_SHARED((NPAD, H), jnp.float32),
            pltpu.SemaphoreType.DMA,
            pltpu.SemaphoreType.DMA,
            pltpu.SemaphoreType.DMA,
            pltpu.SemaphoreType.DMA,
        ],
    )
    def k(e_hbm, dst_hbm, pe_hbm, idx0, ebuf0, idx1, ebuf1, zbe, she,
          semi0, seme0, semi1, seme1):
        cid = lax.axis_index("c")
        sid = lax.axis_index("s")
        wid = sid * NC + cid
        bufs = ((idx0, ebuf0, semi0, seme0), (idx1, ebuf1, semi1, seme1))

        def z1(j, c):
            for kk in range(8):
                zbe[j, pl.ds(kk * 16, 16)] = jnp.zeros((16,), jnp.float32)
            return c

        lax.fori_loop(0, 160, z1, 0)

        for t in range(NPT // 160):
            pltpu.sync_copy(zbe, she.at[pl.ds(sid * NPT + t * 160, 160)])
        plsc.subcore_barrier()

        base0 = wid * EPW

        def load_start(i, b):
            idx, ebuf, semi, seme = bufs[b]
            base = base0 + i * C
            pltpu.async_copy(dst_hbm.at[pl.ds(base, C)], idx, semi)
            pltpu.async_copy(e_hbm.at[pl.ds(base, C)], ebuf, seme)

        def do_scatter(b):
            idx, ebuf, semi, seme = bufs[b]
            pltpu.make_async_copy(dst_hbm.at[pl.ds(0, C)], idx, semi).wait()
            pltpu.make_async_copy(e_hbm.at[pl.ds(0, C)], ebuf, seme).wait()
            pltpu.sync_copy(ebuf, she.at[idx], add=True)

        load_start(0, 0)
        load_start(1, 1)

        def pair(p, carry):
            do_scatter(0)
            load_start(2 * p + 2, 0)
            do_scatter(1)

            @pl.when(2 * p + 3 < NCHUNK)
            def _():
                load_start(2 * p + 3, 1)

            return carry

        lax.fori_loop(0, (NCHUNK - 1) // 2, pair, 0)
        do_scatter(0)
        plsc.subcore_barrier()

        sl = pl.ds(sid * NPT, NPT)
        pltpu.sync_copy(she.at[sl], pe_hbm.at[cid, sl])

    return k(e_new, dstv)


def _sc_count(dstv):
    mesh = plsc.VectorSubcoreMesh(core_axis_name="c", subcore_axis_name="s")

    @functools.partial(
        pl.kernel,
        mesh=mesh,
        out_type=jax.ShapeDtypeStruct((NC, NPAD, H), jnp.float32),
        scratch_types=[
            pltpu.VMEM((C,), jnp.int32),
            pltpu.VMEM((C, H), jnp.float32),
            pltpu.VMEM((160, H), jnp.float32),
            pltpu.VMEM_SHARED((NPAD, H), jnp.float32),
        ],
    )
    def k(dst_hbm, pc_hbm, idx, ones, zbe, shc):
        cid = lax.axis_index("c")
        sid = lax.axis_index("s")
        wid = sid * NC + cid

        def z1(j, c):
            for kk in range(8):
                zbe[j, pl.ds(kk * 16, 16)] = jnp.zeros((16,), jnp.float32)
            return c

        lax.fori_loop(0, 160, z1, 0)

        def o1(j, c):
            for kk in range(8):
                ones[j, pl.ds(kk * 16, 16)] = jnp.ones((16,), jnp.float32)
            return c

        lax.fori_loop(0, C, o1, 0)

        for t in range(NPT // 160):
            pltpu.sync_copy(zbe, shc.at[pl.ds(sid * NPT + t * 160, 160)])
        plsc.subcore_barrier()

        base0 = wid * EPW

        def chunk(i, carry):
            base = base0 + i * C
            pltpu.sync_copy(dst_hbm.at[pl.ds(base, C)], idx)
            pltpu.sync_copy(ones, shc.at[idx], add=True)
            return carry

        lax.fori_loop(0, NCHUNK, chunk, 0)
        plsc.subcore_barrier()

        sl = pl.ds(sid * NPT, NPT)
        pltpu.sync_copy(shc.at[sl], pc_hbm.at[cid, sl])

    return k(dstv)


def kernel(node_attr, edge_attr, edge_index, params):
    src = edge_index[0]
    dst = edge_index[1]

    v = _mlp2(node_attr, *params['enc_node'], True, 1000)
    e = _mlp2(edge_attr, *params['enc_edge'], True, 2000)
    parts_c = _sc_count(dst)

    for i in range(STEPS):
        w0, b0, w1, b1 = params['edge_mlps'][i]
        w0e, w0d, w0s = w0[0:H], w0[H:2 * H], w0[2 * H:3 * H]
        pd, ps = _vp(v, w0d, w0s)
        g = _sc_gather_add(pd, ps, dst, src)
        e_new, e_next = _edge_mlp(e, g, w0e, b0, w1, b1, i < STEPS - 1)
        parts_e = _sc_scatter_add(e_new, dst)
        nw0, nb0, nw1, nb1 = params['node_mlps'][i]
        v = _node_mlp(parts_e, parts_c, v, nw0[0:H], nw0[H:2 * H], nb0, nw1, nb1)
        e = e_next

    return _mlp2(v, *params['dec'], False, 1000)


# speedup vs baseline: 1.0174x; 1.0174x over previous
import functools

import jax
import jax.numpy as jnp
from jax import lax
from jax.experimental import pallas as pl
from jax.experimental.pallas import tpu as pltpu
from jax.experimental.pallas import tpu_sc as plsc

N = 10000
E = 320000
H = 128
STEPS = 3

NC = 2
NS = 16
NW = NC * NS
EPW = E // NW
C = 80
NCHUNK = EPW // C
NPAD = 10240
NPT = NPAD // NS


def _mlp2_body(act_last, proj, x_ref, w0_ref, b0_ref, w1_ref, b1_ref, *rest):
    h = jnp.maximum(
        jnp.dot(x_ref[...], w0_ref[...], preferred_element_type=jnp.float32)
        + b0_ref[...], 0.0)
    y = jnp.dot(h, w1_ref[...], preferred_element_type=jnp.float32) + b1_ref[...]
    y = jnp.maximum(y, 0.0) if act_last else y
    if proj:
        wd_ref, ws_ref, o_ref, pd_ref, ps_ref = rest
        o_ref[...] = y
        pd_ref[...] = jnp.dot(y, wd_ref[...], preferred_element_type=jnp.float32)
        ps_ref[...] = jnp.dot(y, ws_ref[...], preferred_element_type=jnp.float32)
    else:
        rest[0][...] = y


def _mlp2(x, w0, b0, w1, b1, act_last, blk, proj=None):
    n, d = x.shape
    h = w0.shape[1]
    in_specs = [
        pl.BlockSpec((blk, d), lambda i: (i, 0)),
        pl.BlockSpec((d, h), lambda i: (0, 0)),
        pl.BlockSpec((1, h), lambda i: (0, 0)),
        pl.BlockSpec((h, h), lambda i: (0, 0)),
        pl.BlockSpec((1, h), lambda i: (0, 0)),
    ]
    args = [x, w0, b0.reshape(1, h), w1, b1.reshape(1, h)]
    n_out = 1
    if proj is not None:
        in_specs += [pl.BlockSpec((h, h), lambda i: (0, 0))] * 2
        args += [proj[0], proj[1]]
        n_out = 3
    out = pl.pallas_call(
        functools.partial(_mlp2_body, act_last, proj is not None),
        grid=(n // blk,),
        in_specs=in_specs,
        out_specs=[pl.BlockSpec((blk, h), lambda i: (i, 0))] * n_out,
        out_shape=[jax.ShapeDtypeStruct((n, h), jnp.float32)] * n_out,
    )(*args)
    return out if proj is not None else out[0]


def _edge_body(with_residual, e_ref, g_ref, w0_ref, b0_ref, w1_ref, b1_ref,
               en_ref, eo_ref=None):
    e = e_ref[...]
    h = jnp.maximum(
        jnp.dot(e, w0_ref[...], preferred_element_type=jnp.float32)
        + g_ref[...] + b0_ref[...], 0.0)
    en = jnp.maximum(
        jnp.dot(h, w1_ref[...], preferred_element_type=jnp.float32)
        + b1_ref[...], 0.0)
    en_ref[...] = en
    if with_residual:
        eo_ref[...] = en + e


def _edge_mlp(e, g, w0e, b0, w1, b1, with_residual, blk=2000):
    n_out = 2 if with_residual else 1
    out = pl.pallas_call(
        functools.partial(_edge_body, with_residual),
        grid=(E // blk,),
        in_specs=[
            pl.BlockSpec((blk, H), lambda i: (i, 0)),
            pl.BlockSpec((blk, H), lambda i: (i, 0)),
            pl.BlockSpec((H, H), lambda i: (0, 0)),
            pl.BlockSpec((1, H), lambda i: (0, 0)),
            pl.BlockSpec((H, H), lambda i: (0, 0)),
            pl.BlockSpec((1, H), lambda i: (0, 0)),
        ],
        out_specs=[pl.BlockSpec((blk, H), lambda i: (i, 0))] * n_out,
        out_shape=[jax.ShapeDtypeStruct((E, H), jnp.float32)] * n_out,
    )(e, g, w0e, b0.reshape(1, H), w1, b1.reshape(1, H))
    return out if with_residual else (out[0], None)


def _node_body(proj, pe_ref, pc_ref, v_ref, w0a_ref, w0v_ref, b0_ref, w1_ref,
               b1_ref, *rest):
    s_e = pe_ref[0, :, :] + pe_ref[1, :, :]
    cnt = pc_ref[0, :, 0:1] + pc_ref[1, :, 0:1]
    agg = s_e * (1.0 / jnp.maximum(cnt, 1.0))
    v = v_ref[...]
    h = jnp.maximum(
        jnp.dot(agg, w0a_ref[...], preferred_element_type=jnp.float32)
        + jnp.dot(v, w0v_ref[...], preferred_element_type=jnp.float32)
        + b0_ref[...], 0.0)
    vn = jnp.maximum(
        jnp.dot(h, w1_ref[...], preferred_element_type=jnp.float32)
        + b1_ref[...], 0.0)
    vo = vn + v
    if proj:
        wd_ref, ws_ref, vo_ref, pd_ref, ps_ref = rest
        vo_ref[...] = vo
        pd_ref[...] = jnp.dot(vo, wd_ref[...],
                              preferred_element_type=jnp.float32)
        ps_ref[...] = jnp.dot(vo, ws_ref[...],
                              preferred_element_type=jnp.float32)
    else:
        rest[0][...] = vo


def _node_mlp(parts_e, parts_c, v, w0a, w0v, b0, w1, b1, proj=None, blk=1000):
    in_specs = [
        pl.BlockSpec((NC, blk, H), lambda i: (0, i, 0)),
        pl.BlockSpec((NC, blk, H), lambda i: (0, i, 0)),
        pl.BlockSpec((blk, H), lambda i: (i, 0)),
        pl.BlockSpec((H, H), lambda i: (0, 0)),
        pl.BlockSpec((H, H), lambda i: (0, 0)),
        pl.BlockSpec((1, H), lambda i: (0, 0)),
        pl.BlockSpec((H, H), lambda i: (0, 0)),
        pl.BlockSpec((1, H), lambda i: (0, 0)),
    ]
    args = [parts_e, parts_c, v, w0a, w0v, b0.reshape(1, H), w1,
            b1.reshape(1, H)]
    n_out = 1
    if proj is not None:
        in_specs += [pl.BlockSpec((H, H), lambda i: (0, 0))] * 2
        args += [proj[0], proj[1]]
        n_out = 3
    out = pl.pallas_call(
        functools.partial(_node_body, proj is not None),
        grid=(N // blk,),
        in_specs=in_specs,
        out_specs=[pl.BlockSpec((blk, H), lambda i: (i, 0))] * n_out,
        out_shape=[jax.ShapeDtypeStruct((N, H), jnp.float32)] * n_out,
    )(*args)
    return out if proj is not None else out[0]


def _sc_gather_add(pd, ps, dstv, srcv):
    mesh = plsc.VectorSubcoreMesh(core_axis_name="c", subcore_axis_name="s")

    @functools.partial(
        pl.kernel,
        mesh=mesh,
        out_type=jax.ShapeDtypeStruct((E, H), jnp.float32),
        scratch_types=[
            pltpu.VMEM((C,), jnp.int32),
            pltpu.VMEM((C,), jnp.int32),
            pltpu.VMEM((C,), jnp.int32),
            pltpu.VMEM((C,), jnp.int32),
            pltpu.VMEM((C, H), jnp.float32),
            pltpu.VMEM((C, H), jnp.float32),
            pltpu.VMEM((C, H), jnp.float32),
            pltpu.VMEM((C, H), jnp.float32),
            pltpu.SemaphoreType.DMA,
            pltpu.SemaphoreType.DMA,
            pltpu.SemaphoreType.DMA,
            pltpu.SemaphoreType.DMA,
            pltpu.SemaphoreType.DMA,
            pltpu.SemaphoreType.DMA,
            pltpu.SemaphoreType.DMA,
            pltpu.SemaphoreType.DMA,
        ],
    )
    def k(pd_hbm, ps_hbm, dst_hbm, src_hbm, g_hbm,
          idxd0, idxs0, idxd1, idxs1, rd0, rs0, rd1, rs1,
          semd0, sems0, semd1, sems1, semi0, semi1, semo0, semo1):
        wid = lax.axis_index("s") * NC + lax.axis_index("c")
        base0 = wid * EPW
        bufs = ((idxd0, idxs0, rd0, rs0, semd0, sems0, semi0, semo0),
                (idxd1, idxs1, rd1, rs1, semd1, sems1, semi1, semo1))

        def idx_start(i, b):
            idxd, idxs = bufs[b][0], bufs[b][1]
            semi = bufs[b][6]
            base = base0 + i * C
            pltpu.async_copy(dst_hbm.at[pl.ds(base, C)], idxd, semi)
            pltpu.async_copy(src_hbm.at[pl.ds(base, C)], idxs, semi)

        def gather_start(b, first):
            idxd, idxs, rd, rs, semd, sems_, semi, semo = bufs[b]
            pltpu.make_async_copy(dst_hbm.at[pl.ds(0, C)], idxd, semi).wait()
            pltpu.make_async_copy(src_hbm.at[pl.ds(0, C)], idxs, semi).wait()
            if not first:
                pltpu.make_async_copy(rd, g_hbm.at[pl.ds(0, C)], semo).wait()
            pltpu.async_copy(pd_hbm.at[idxd], rd, semd)
            pltpu.async_copy(ps_hbm.at[idxs], rs, sems_)

        def finish(i, b):
            idxd, idxs, rd, rs, semd, sems_, _, semo = bufs[b]
            pltpu.make_async_copy(pd_hbm.at[idxd], rd, semd).wait()
            pltpu.make_async_copy(ps_hbm.at[idxs], rs, sems_).wait()

            def row(j, c2):
                for kk in range(8):
                    sl = pl.ds(kk * 16, 16)
                    rd[j, sl] = rd[j, sl] + rs[j, sl]
                return c2

            lax.fori_loop(0, C, row, 0)
            pltpu.async_copy(rd, g_hbm.at[pl.ds(base0 + i * C, C)], semo)

        idx_start(0, 0)
        gather_start(0, True)
        idx_start(1, 1)
        gather_start(1, True)

        def pair(p, carry):
            finish(2 * p, 0)
            idx_start(2 * p + 2, 0)
            gather_start(0, False)
            finish(2 * p + 1, 1)

            @pl.when(2 * p + 3 < NCHUNK)
            def _():
                idx_start(2 * p + 3, 1)
                gather_start(1, False)

            return carry

        lax.fori_loop(0, (NCHUNK - 1) // 2, pair, 0)
        finish(NCHUNK - 1, 0)
        pltpu.make_async_copy(rd0, g_hbm.at[pl.ds(0, C)], semo0).wait()
        pltpu.make_async_copy(rd1, g_hbm.at[pl.ds(0, C)], semo1).wait()

    return k(pd, ps, dstv, srcv)


def _sc_scatter_add(e_new, dstv):
    mesh = plsc.VectorSubcoreMesh(core_axis_name="c", subcore_axis_name="s")

    @functools.partial(
        pl.kernel,
        mesh=mesh,
        out_type=jax.ShapeDtypeStruct((NC, NPAD, H), jnp.float32),
        scratch_types=[
            pltpu.VMEM((C,), jnp.int32),
            pltpu.VMEM((C, H), jnp.float32),
            pltpu.VMEM((C,), jnp.int32),
            pltpu.VMEM((C, H), jnp.float32),
            pltpu.VMEM((160, H), jnp.float32),
            pltpu.VMEM_SHARED((NPAD, H), jnp.float32),
            pltpu.SemaphoreType.DMA,
            pltpu.SemaphoreType.DMA,
            pltpu.SemaphoreType.DMA,
            pltpu.SemaphoreType.DMA,
        ],
    )
    def k(e_hbm, dst_hbm, pe_hbm, idx0, ebuf0, idx1, ebuf1, zbe, she,
          semi0, seme0, semi1, seme1):
        cid = lax.axis_index("c")
        sid = lax.axis_index("s")
        wid = sid * NC + cid
        bufs = ((idx0, ebuf0, semi0, seme0), (idx1, ebuf1, semi1, seme1))

        def z1(j, c):
            for kk in range(8):
                zbe[j, pl.ds(kk * 16, 16)] = jnp.zeros((16,), jnp.float32)
            return c

        lax.fori_loop(0, 160, z1, 0)

        for t in range(NPT // 160):
            pltpu.sync_copy(zbe, she.at[pl.ds(sid * NPT + t * 160, 160)])
        plsc.subcore_barrier()

        base0 = wid * EPW

        def load_start(i, b):
            idx, ebuf, semi, seme = bufs[b]
            base = base0 + i * C
            pltpu.async_copy(dst_hbm.at[pl.ds(base, C)], idx, semi)
            pltpu.async_copy(e_hbm.at[pl.ds(base, C)], ebuf, seme)

        def do_scatter(b):
            idx, ebuf, semi, seme = bufs[b]
            pltpu.make_async_copy(dst_hbm.at[pl.ds(0, C)], idx, semi).wait()
            pltpu.make_async_copy(e_hbm.at[pl.ds(0, C)], ebuf, seme).wait()
            pltpu.sync_copy(ebuf, she.at[idx], add=True)

        load_start(0, 0)
        load_start(1, 1)

        def pair(p, carry):
            do_scatter(0)
            load_start(2 * p + 2, 0)
            do_scatter(1)

            @pl.when(2 * p + 3 < NCHUNK)
            def _():
                load_start(2 * p + 3, 1)

            return carry

        lax.fori_loop(0, (NCHUNK - 1) // 2, pair, 0)
        do_scatter(0)
        plsc.subcore_barrier()

        sl = pl.ds(sid * NPT, NPT)
        pltpu.sync_copy(she.at[sl], pe_hbm.at[cid, sl])

    return k(e_new, dstv)


def _sc_count(dstv):
    mesh = plsc.VectorSubcoreMesh(core_axis_name="c", subcore_axis_name="s")

    @functools.partial(
        pl.kernel,
        mesh=mesh,
        out_type=jax.ShapeDtypeStruct((NC, NPAD, H), jnp.float32),
        scratch_types=[
            pltpu.VMEM((C,), jnp.int32),
            pltpu.VMEM((C,), jnp.int32),
            pltpu.VMEM((C, H), jnp.float32),
            pltpu.VMEM((160, H), jnp.float32),
            pltpu.VMEM_SHARED((NPAD, H), jnp.float32),
            pltpu.SemaphoreType.DMA,
            pltpu.SemaphoreType.DMA,
        ],
    )
    def k(dst_hbm, pc_hbm, idx0, idx1, ones, zbe, shc, semi0, semi1):
        cid = lax.axis_index("c")
        sid = lax.axis_index("s")
        wid = sid * NC + cid

        def z1(j, c):
            for kk in range(8):
                zbe[j, pl.ds(kk * 16, 16)] = jnp.zeros((16,), jnp.float32)
            return c

        lax.fori_loop(0, 160, z1, 0)

        def o1(j, c):
            for kk in range(8):
                ones[j, pl.ds(kk * 16, 16)] = jnp.ones((16,), jnp.float32)
            return c

        lax.fori_loop(0, C, o1, 0)

        for t in range(NPT // 160):
            pltpu.sync_copy(zbe, shc.at[pl.ds(sid * NPT + t * 160, 160)])
        plsc.subcore_barrier()

        base0 = wid * EPW
        bufs = ((idx0, semi0), (idx1, semi1))

        def load_start(i, b):
            idx, semi = bufs[b]
            pltpu.async_copy(dst_hbm.at[pl.ds(base0 + i * C, C)], idx, semi)

        def do_scatter(b):
            idx, semi = bufs[b]
            pltpu.make_async_copy(dst_hbm.at[pl.ds(0, C)], idx, semi).wait()
            pltpu.sync_copy(ones, shc.at[idx], add=True)

        load_start(0, 0)
        load_start(1, 1)

        def pair(p, carry):
            do_scatter(0)
            load_start(2 * p + 2, 0)
            do_scatter(1)

            @pl.when(2 * p + 3 < NCHUNK)
            def _():
                load_start(2 * p + 3, 1)

            return carry

        lax.fori_loop(0, (NCHUNK - 1) // 2, pair, 0)
        do_scatter(0)
        plsc.subcore_barrier()

        sl = pl.ds(sid * NPT, NPT)
        pltpu.sync_copy(shc.at[sl], pc_hbm.at[cid, sl])

    return k(dstv)


def kernel(node_attr, edge_attr, edge_index, params):
    src = edge_index[0]
    dst = edge_index[1]

    def edge_w(i):
        w0 = params['edge_mlps'][i][0]
        return w0[0:H], w0[H:2 * H], w0[2 * H:3 * H]

    w0e, w0d, w0s = edge_w(0)
    v, pd, ps = _mlp2(node_attr, *params['enc_node'], True, 1000,
                      proj=(w0d, w0s))
    e = _mlp2(edge_attr, *params['enc_edge'], True, 2000)
    parts_c = _sc_count(dst)

    for i in range(STEPS):
        _, b0, w1, b1 = params['edge_mlps'][i]
        g = _sc_gather_add(pd, ps, dst, src)
        e_new, e_next = _edge_mlp(e, g, w0e, b0, w1, b1, i < STEPS - 1)
        parts_e = _sc_scatter_add(e_new, dst)
        nw0, nb0, nw1, nb1 = params['node_mlps'][i]
        if i < STEPS - 1:
            w0e, w0d, w0s = edge_w(i + 1)
            v, pd, ps = _node_mlp(parts_e, parts_c, v, nw0[0:H], nw0[H:2 * H],
                                  nb0, nw1, nb1, proj=(w0d, w0s))
        else:
            v = _node_mlp(parts_e, parts_c, v, nw0[0:H], nw0[H:2 * H], nb0,
                          nw1, nb1)
        e = e_next

    return _mlp2(v, *params['dec'], False, 1000)


# speedup vs baseline: 1.0650x; 1.0468x over previous
import functools

import jax
import jax.numpy as jnp
from jax import lax
from jax.experimental import pallas as pl
from jax.experimental.pallas import tpu as pltpu
from jax.experimental.pallas import tpu_sc as plsc

N = 10000
E = 320000
H = 128
STEPS = 3

NC = 2
NS = 16
NW = NC * NS
EPW = E // NW
C = 80
NCHUNK = EPW // C
NPAD = 10240
NPT = NPAD // NS


def _mlp2_body(act_last, proj, x_ref, w0_ref, b0_ref, w1_ref, b1_ref, *rest):
    h = jnp.maximum(
        jnp.dot(x_ref[...], w0_ref[...], preferred_element_type=jnp.float32)
        + b0_ref[...], 0.0)
    y = jnp.dot(h, w1_ref[...], preferred_element_type=jnp.float32) + b1_ref[...]
    y = jnp.maximum(y, 0.0) if act_last else y
    y = y.astype(rest[0].dtype)
    if proj:
        wd_ref, ws_ref, o_ref, pd_ref, ps_ref = rest
        o_ref[...] = y
        pd_ref[...] = jnp.dot(y, wd_ref[...], preferred_element_type=jnp.float32)
        ps_ref[...] = jnp.dot(y, ws_ref[...], preferred_element_type=jnp.float32)
    else:
        rest[0][...] = y


def _mlp2(x, w0, b0, w1, b1, act_last, blk, proj=None,
          out_dtype=jnp.float32):
    n, d = x.shape
    h = w0.shape[1]
    in_specs = [
        pl.BlockSpec((blk, d), lambda i: (i, 0)),
        pl.BlockSpec((d, h), lambda i: (0, 0)),
        pl.BlockSpec((1, h), lambda i: (0, 0)),
        pl.BlockSpec((h, h), lambda i: (0, 0)),
        pl.BlockSpec((1, h), lambda i: (0, 0)),
    ]
    args = [x, w0, b0.reshape(1, h), w1, b1.reshape(1, h)]
    n_out = 1
    if proj is not None:
        in_specs += [pl.BlockSpec((h, h), lambda i: (0, 0))] * 2
        args += [proj[0], proj[1]]
        n_out = 3
    out = pl.pallas_call(
        functools.partial(_mlp2_body, act_last, proj is not None),
        grid=(n // blk,),
        in_specs=in_specs,
        out_specs=[pl.BlockSpec((blk, h), lambda i: (i, 0))] * n_out,
        out_shape=[jax.ShapeDtypeStruct((n, h), out_dtype)] * n_out,
    )(*args)
    return out if proj is not None else out[0]


def _edge_body(with_residual, e_ref, g_ref, w0_ref, b0_ref, w1_ref, b1_ref,
               en_ref, eo_ref=None):
    e = e_ref[...].astype(jnp.float32)
    h = jnp.maximum(
        jnp.dot(e, w0_ref[...], preferred_element_type=jnp.float32)
        + g_ref[...] + b0_ref[...], 0.0)
    en = jnp.maximum(
        jnp.dot(h, w1_ref[...], preferred_element_type=jnp.float32)
        + b1_ref[...], 0.0)
    en_ref[...] = en
    if with_residual:
        eo_ref[...] = (en + e).astype(jnp.bfloat16)


def _edge_mlp(e, g, w0e, b0, w1, b1, with_residual, blk=2000):
    n_out = 2 if with_residual else 1
    out = pl.pallas_call(
        functools.partial(_edge_body, with_residual),
        grid=(E // blk,),
        in_specs=[
            pl.BlockSpec((blk, H), lambda i: (i, 0)),
            pl.BlockSpec((blk, H), lambda i: (i, 0)),
            pl.BlockSpec((H, H), lambda i: (0, 0)),
            pl.BlockSpec((1, H), lambda i: (0, 0)),
            pl.BlockSpec((H, H), lambda i: (0, 0)),
            pl.BlockSpec((1, H), lambda i: (0, 0)),
        ],
        out_specs=[pl.BlockSpec((blk, H), lambda i: (i, 0))] * n_out,
        out_shape=[jax.ShapeDtypeStruct((E, H), jnp.float32),
                   jax.ShapeDtypeStruct((E, H), jnp.bfloat16)][:n_out],
    )(e, g, w0e, b0.reshape(1, H), w1, b1.reshape(1, H))
    return out if with_residual else (out[0], None)


def _node_body(proj, pe_ref, pc_ref, v_ref, w0a_ref, w0v_ref, b0_ref, w1_ref,
               b1_ref, *rest):
    s_e = pe_ref[0, :, :] + pe_ref[1, :, :]
    cnt = pc_ref[0, :, 0:1] + pc_ref[1, :, 0:1]
    agg = s_e * (1.0 / jnp.maximum(cnt, 1.0))
    v = v_ref[...]
    h = jnp.maximum(
        jnp.dot(agg, w0a_ref[...], preferred_element_type=jnp.float32)
        + jnp.dot(v, w0v_ref[...], preferred_element_type=jnp.float32)
        + b0_ref[...], 0.0)
    vn = jnp.maximum(
        jnp.dot(h, w1_ref[...], preferred_element_type=jnp.float32)
        + b1_ref[...], 0.0)
    vo = vn + v
    if proj:
        wd_ref, ws_ref, vo_ref, pd_ref, ps_ref = rest
        vo_ref[...] = vo
        pd_ref[...] = jnp.dot(vo, wd_ref[...],
                              preferred_element_type=jnp.float32)
        ps_ref[...] = jnp.dot(vo, ws_ref[...],
                              preferred_element_type=jnp.float32)
    else:
        rest[0][...] = vo


def _node_mlp(parts_e, parts_c, v, w0a, w0v, b0, w1, b1, proj=None, blk=1000):
    in_specs = [
        pl.BlockSpec((NC, blk, H), lambda i: (0, i, 0)),
        pl.BlockSpec((NC, blk, H), lambda i: (0, i, 0)),
        pl.BlockSpec((blk, H), lambda i: (i, 0)),
        pl.BlockSpec((H, H), lambda i: (0, 0)),
        pl.BlockSpec((H, H), lambda i: (0, 0)),
        pl.BlockSpec((1, H), lambda i: (0, 0)),
        pl.BlockSpec((H, H), lambda i: (0, 0)),
        pl.BlockSpec((1, H), lambda i: (0, 0)),
    ]
    args = [parts_e, parts_c, v, w0a, w0v, b0.reshape(1, H), w1,
            b1.reshape(1, H)]
    n_out = 1
    if proj is not None:
        in_specs += [pl.BlockSpec((H, H), lambda i: (0, 0))] * 2
        args += [proj[0], proj[1]]
        n_out = 3
    out = pl.pallas_call(
        functools.partial(_node_body, proj is not None),
        grid=(N // blk,),
        in_specs=in_specs,
        out_specs=[pl.BlockSpec((blk, H), lambda i: (i, 0))] * n_out,
        out_shape=[jax.ShapeDtypeStruct((N, H), jnp.float32)] * n_out,
    )(*args)
    return out if proj is not None else out[0]


def _sc_gather_add(pd, ps, dstv, srcv):
    mesh = plsc.VectorSubcoreMesh(core_axis_name="c", subcore_axis_name="s")

    @functools.partial(
        pl.kernel,
        mesh=mesh,
        out_type=jax.ShapeDtypeStruct((E, H), jnp.float32),
        scratch_types=[
            pltpu.VMEM((C,), jnp.int32),
            pltpu.VMEM((C,), jnp.int32),
            pltpu.VMEM((C,), jnp.int32),
            pltpu.VMEM((C,), jnp.int32),
            pltpu.VMEM((C, H), jnp.float32),
            pltpu.VMEM((C, H), jnp.float32),
            pltpu.VMEM((C, H), jnp.float32),
            pltpu.VMEM((C, H), jnp.float32),
            pltpu.SemaphoreType.DMA,
            pltpu.SemaphoreType.DMA,
            pltpu.SemaphoreType.DMA,
            pltpu.SemaphoreType.DMA,
            pltpu.SemaphoreType.DMA,
            pltpu.SemaphoreType.DMA,
            pltpu.SemaphoreType.DMA,
            pltpu.SemaphoreType.DMA,
        ],
    )
    def k(pd_hbm, ps_hbm, dst_hbm, src_hbm, g_hbm,
          idxd0, idxs0, idxd1, idxs1, rd0, rs0, rd1, rs1,
          semd0, sems0, semd1, sems1, semi0, semi1, semo0, semo1):
        wid = lax.axis_index("s") * NC + lax.axis_index("c")
        base0 = wid * EPW
        bufs = ((idxd0, idxs0, rd0, rs0, semd0, sems0, semi0, semo0),
                (idxd1, idxs1, rd1, rs1, semd1, sems1, semi1, semo1))

        def idx_start(i, b):
            idxd, idxs = bufs[b][0], bufs[b][1]
            semi = bufs[b][6]
            base = base0 + i * C
            pltpu.async_copy(dst_hbm.at[pl.ds(base, C)], idxd, semi)
            pltpu.async_copy(src_hbm.at[pl.ds(base, C)], idxs, semi)

        def gather_start(b, first):
            idxd, idxs, rd, rs, semd, sems_, semi, semo = bufs[b]
            pltpu.make_async_copy(dst_hbm.at[pl.ds(0, C)], idxd, semi).wait()
            pltpu.make_async_copy(src_hbm.at[pl.ds(0, C)], idxs, semi).wait()
            if not first:
                pltpu.make_async_copy(rd, g_hbm.at[pl.ds(0, C)], semo).wait()
            pltpu.async_copy(pd_hbm.at[idxd], rd, semd)
            pltpu.async_copy(ps_hbm.at[idxs], rs, sems_)

        def finish(i, b):
            idxd, idxs, rd, rs, semd, sems_, _, semo = bufs[b]
            pltpu.make_async_copy(pd_hbm.at[idxd], rd, semd).wait()
            pltpu.make_async_copy(ps_hbm.at[idxs], rs, sems_).wait()

            def row(j, c2):
                for kk in range(8):
                    sl = pl.ds(kk * 16, 16)
                    rd[j, sl] = rd[j, sl] + rs[j, sl]
                return c2

            lax.fori_loop(0, C, row, 0)
            pltpu.async_copy(rd, g_hbm.at[pl.ds(base0 + i * C, C)], semo)

        idx_start(0, 0)
        gather_start(0, True)
        idx_start(1, 1)
        gather_start(1, True)

        def pair(p, carry):
            finish(2 * p, 0)
            idx_start(2 * p + 2, 0)
            gather_start(0, False)
            finish(2 * p + 1, 1)

            @pl.when(2 * p + 3 < NCHUNK)
            def _():
                idx_start(2 * p + 3, 1)
                gather_start(1, False)

            return carry

        lax.fori_loop(0, (NCHUNK - 1) // 2, pair, 0)
        finish(NCHUNK - 1, 0)
        pltpu.make_async_copy(rd0, g_hbm.at[pl.ds(0, C)], semo0).wait()
        pltpu.make_async_copy(rd1, g_hbm.at[pl.ds(0, C)], semo1).wait()

    return k(pd, ps, dstv, srcv)


def _sc_scatter_add(e_new, dstv):
    mesh = plsc.VectorSubcoreMesh(core_axis_name="c", subcore_axis_name="s")

    @functools.partial(
        pl.kernel,
        mesh=mesh,
        out_type=jax.ShapeDtypeStruct((NC, NPAD, H), jnp.float32),
        scratch_types=[
            pltpu.VMEM((C,), jnp.int32),
            pltpu.VMEM((C, H), jnp.float32),
            pltpu.VMEM((C,), jnp.int32),
            pltpu.VMEM((C, H), jnp.float32),
            pltpu.VMEM((160, H), jnp.float32),
            pltpu.VMEM_SHARED((NPAD, H), jnp.float32),
            pltpu.SemaphoreType.DMA,
            pltpu.SemaphoreType.DMA,
            pltpu.SemaphoreType.DMA,
            pltpu.SemaphoreType.DMA,
        ],
    )
    def k(e_hbm, dst_hbm, pe_hbm, idx0, ebuf0, idx1, ebuf1, zbe, she,
          semi0, seme0, semi1, seme1):
        cid = lax.axis_index("c")
        sid = lax.axis_index("s")
        wid = sid * NC + cid
        bufs = ((idx0, ebuf0, semi0, seme0), (idx1, ebuf1, semi1, seme1))

        def z1(j, c):
            for kk in range(8):
                zbe[j, pl.ds(kk * 16, 16)] = jnp.zeros((16,), jnp.float32)
            return c

        lax.fori_loop(0, 160, z1, 0)

        for t in range(NPT // 160):
            pltpu.sync_copy(zbe, she.at[pl.ds(sid * NPT + t * 160, 160)])
        plsc.subcore_barrier()

        base0 = wid * EPW

        def load_start(i, b):
            idx, ebuf, semi, seme = bufs[b]
            base = base0 + i * C
            pltpu.async_copy(dst_hbm.at[pl.ds(base, C)], idx, semi)
            pltpu.async_copy(e_hbm.at[pl.ds(base, C)], ebuf, seme)

        def do_scatter(b):
            idx, ebuf, semi, seme = bufs[b]
            pltpu.make_async_copy(dst_hbm.at[pl.ds(0, C)], idx, semi).wait()
            pltpu.make_async_copy(e_hbm.at[pl.ds(0, C)], ebuf, seme).wait()
            pltpu.sync_copy(ebuf, she.at[idx], add=True)

        load_start(0, 0)
        load_start(1, 1)

        def pair(p, carry):
            do_scatter(0)
            load_start(2 * p + 2, 0)
            do_scatter(1)

            @pl.when(2 * p + 3 < NCHUNK)
            def _():
                load_start(2 * p + 3, 1)

            return carry

        lax.fori_loop(0, (NCHUNK - 1) // 2, pair, 0)
        do_scatter(0)
        plsc.subcore_barrier()

        sl = pl.ds(sid * NPT, NPT)
        pltpu.sync_copy(she.at[sl], pe_hbm.at[cid, sl])

    return k(e_new, dstv)


def _sc_count(dstv):
    mesh = plsc.VectorSubcoreMesh(core_axis_name="c", subcore_axis_name="s")

    @functools.partial(
        pl.kernel,
        mesh=mesh,
        out_type=jax.ShapeDtypeStruct((NC, NPAD, H), jnp.float32),
        scratch_types=[
            pltpu.VMEM((C,), jnp.int32),
            pltpu.VMEM((C,), jnp.int32),
            pltpu.VMEM((C, H), jnp.float32),
            pltpu.VMEM((160, H), jnp.float32),
            pltpu.VMEM_SHARED((NPAD, H), jnp.float32),
            pltpu.SemaphoreType.DMA,
            pltpu.SemaphoreType.DMA,
        ],
    )
    def k(dst_hbm, pc_hbm, idx0, idx1, ones, zbe, shc, semi0, semi1):
        cid = lax.axis_index("c")
        sid = lax.axis_index("s")
        wid = sid * NC + cid

        def z1(j, c):
            for kk in range(8):
                zbe[j, pl.ds(kk * 16, 16)] = jnp.zeros((16,), jnp.float32)
            return c

        lax.fori_loop(0, 160, z1, 0)

        def o1(j, c):
            for kk in range(8):
                ones[j, pl.ds(kk * 16, 16)] = jnp.ones((16,), jnp.float32)
            return c

        lax.fori_loop(0, C, o1, 0)

        for t in range(NPT // 160):
            pltpu.sync_copy(zbe, shc.at[pl.ds(sid * NPT + t * 160, 160)])
        plsc.subcore_barrier()

        base0 = wid * EPW
        bufs = ((idx0, semi0), (idx1, semi1))

        def load_start(i, b):
            idx, semi = bufs[b]
            pltpu.async_copy(dst_hbm.at[pl.ds(base0 + i * C, C)], idx, semi)

        def do_scatter(b):
            idx, semi = bufs[b]
            pltpu.make_async_copy(dst_hbm.at[pl.ds(0, C)], idx, semi).wait()
            pltpu.sync_copy(ones, shc.at[idx], add=True)

        load_start(0, 0)
        load_start(1, 1)

        def pair(p, carry):
            do_scatter(0)
            load_start(2 * p + 2, 0)
            do_scatter(1)

            @pl.when(2 * p + 3 < NCHUNK)
            def _():
                load_start(2 * p + 3, 1)

            return carry

        lax.fori_loop(0, (NCHUNK - 1) // 2, pair, 0)
        do_scatter(0)
        plsc.subcore_barrier()

        sl = pl.ds(sid * NPT, NPT)
        pltpu.sync_copy(shc.at[sl], pc_hbm.at[cid, sl])

    return k(dstv)


def kernel(node_attr, edge_attr, edge_index, params):
    src = edge_index[0]
    dst = edge_index[1]

    def edge_w(i):
        w0 = params['edge_mlps'][i][0]
        return w0[0:H], w0[H:2 * H], w0[2 * H:3 * H]

    w0e, w0d, w0s = edge_w(0)
    v, pd, ps = _mlp2(node_attr, *params['enc_node'], True, 1000,
                      proj=(w0d, w0s))
    e = _mlp2(edge_attr, *params['enc_edge'], True, 2000,
              out_dtype=jnp.bfloat16)
    parts_c = _sc_count(dst)

    for i in range(STEPS):
        _, b0, w1, b1 = params['edge_mlps'][i]
        g = _sc_gather_add(pd, ps, dst, src)
        e_new, e_next = _edge_mlp(e, g, w0e, b0, w1, b1, i < STEPS - 1)
        parts_e = _sc_scatter_add(e_new, dst)
        nw0, nb0, nw1, nb1 = params['node_mlps'][i]
        if i < STEPS - 1:
            w0e, w0d, w0s = edge_w(i + 1)
            v, pd, ps = _node_mlp(parts_e, parts_c, v, nw0[0:H], nw0[H:2 * H],
                                  nb0, nw1, nb1, proj=(w0d, w0s))
        else:
            v = _node_mlp(parts_e, parts_c, v, nw0[0:H], nw0[H:2 * H], nb0,
                          nw1, nb1)
        e = e_next

    return _mlp2(v, *params['dec'], False, 1000)


# speedup vs baseline: 1.1694x; 1.0980x over previous
import functools

import jax
import jax.numpy as jnp
from jax import lax
from jax.experimental import pallas as pl
from jax.experimental.pallas import tpu as pltpu
from jax.experimental.pallas import tpu_sc as plsc

N = 10000
E = 320000
H = 128
STEPS = 3

NC = 2
NS = 16
NW = NC * NS
EPW = E // NW
C = 80
NCHUNK = EPW // C
NPAD = 10240
NPT = NPAD // NS


def _mlp2_body(act_last, proj, x_ref, w0_ref, b0_ref, w1_ref, b1_ref, *rest):
    h = jnp.maximum(
        jnp.dot(x_ref[...], w0_ref[...], preferred_element_type=jnp.float32)
        + b0_ref[...], 0.0)
    y = jnp.dot(h, w1_ref[...], preferred_element_type=jnp.float32) + b1_ref[...]
    y = jnp.maximum(y, 0.0) if act_last else y
    y = y.astype(rest[0].dtype)
    if proj:
        wd_ref, ws_ref, o_ref, pd_ref, ps_ref = rest
        o_ref[...] = y
        pd_ref[...] = jnp.dot(y, wd_ref[...], preferred_element_type=jnp.float32)
        ps_ref[...] = jnp.dot(y, ws_ref[...], preferred_element_type=jnp.float32)
    else:
        rest[0][...] = y


def _mlp2(x, w0, b0, w1, b1, act_last, blk, proj=None,
          out_dtype=jnp.float32):
    n, d = x.shape
    h = w0.shape[1]
    in_specs = [
        pl.BlockSpec((blk, d), lambda i: (i, 0)),
        pl.BlockSpec((d, h), lambda i: (0, 0)),
        pl.BlockSpec((1, h), lambda i: (0, 0)),
        pl.BlockSpec((h, h), lambda i: (0, 0)),
        pl.BlockSpec((1, h), lambda i: (0, 0)),
    ]
    args = [x, w0, b0.reshape(1, h), w1, b1.reshape(1, h)]
    n_out = 1
    if proj is not None:
        in_specs += [pl.BlockSpec((h, h), lambda i: (0, 0))] * 2
        args += [proj[0], proj[1]]
        n_out = 3
    out = pl.pallas_call(
        functools.partial(_mlp2_body, act_last, proj is not None),
        grid=(n // blk,),
        in_specs=in_specs,
        out_specs=[pl.BlockSpec((blk, h), lambda i: (i, 0))] * n_out,
        out_shape=[jax.ShapeDtypeStruct((n, h), out_dtype)] * n_out,
    )(*args)
    return out if proj is not None else out[0]


def _edge_body(with_residual, e_ref, g_ref, w0_ref, b0_ref, w1_ref, b1_ref,
               en_ref, eo_ref=None):
    e = e_ref[...].astype(jnp.float32)
    h = jnp.maximum(
        jnp.dot(e, w0_ref[...], preferred_element_type=jnp.float32)
        + g_ref[...] + b0_ref[...], 0.0)
    en = jnp.maximum(
        jnp.dot(h, w1_ref[...], preferred_element_type=jnp.float32)
        + b1_ref[...], 0.0)
    en_ref[...] = en
    if with_residual:
        eo_ref[...] = (en + e).astype(jnp.bfloat16)


def _edge_mlp(e, g, w0e, b0, w1, b1, with_residual, blk=4000):
    n_out = 2 if with_residual else 1
    out = pl.pallas_call(
        functools.partial(_edge_body, with_residual),
        grid=(E // blk,),
        in_specs=[
            pl.BlockSpec((blk, H), lambda i: (i, 0)),
            pl.BlockSpec((blk, H), lambda i: (i, 0)),
            pl.BlockSpec((H, H), lambda i: (0, 0)),
            pl.BlockSpec((1, H), lambda i: (0, 0)),
            pl.BlockSpec((H, H), lambda i: (0, 0)),
            pl.BlockSpec((1, H), lambda i: (0, 0)),
        ],
        out_specs=[pl.BlockSpec((blk, H), lambda i: (i, 0))] * n_out,
        out_shape=[jax.ShapeDtypeStruct((E, H), jnp.float32),
                   jax.ShapeDtypeStruct((E, H), jnp.bfloat16)][:n_out],
    )(e, g, w0e, b0.reshape(1, H), w1, b1.reshape(1, H))
    return out if with_residual else (out[0], None)


def _inv_body(pc_ref, inv_ref):
    cnt = pc_ref[0, :, 0:1] + pc_ref[1, :, 0:1]
    inv_ref[...] = 1.0 / jnp.maximum(cnt, 1.0)


def _inv_cnt(parts_c, blk=1024):
    return pl.pallas_call(
        _inv_body,
        grid=(NPAD // blk,),
        in_specs=[pl.BlockSpec((NC, blk, H), lambda i: (0, i, 0))],
        out_specs=pl.BlockSpec((blk, 1), lambda i: (i, 0)),
        out_shape=jax.ShapeDtypeStruct((NPAD, 1), jnp.float32),
    )(parts_c)


def _node_body(proj, pe_ref, pc_ref, v_ref, w0a_ref, w0v_ref, b0_ref, w1_ref,
               b1_ref, *rest):
    s_e = pe_ref[0, :, :] + pe_ref[1, :, :]
    agg = s_e * pc_ref[...]
    v = v_ref[...]
    h = jnp.maximum(
        jnp.dot(agg, w0a_ref[...], preferred_element_type=jnp.float32)
        + jnp.dot(v, w0v_ref[...], preferred_element_type=jnp.float32)
        + b0_ref[...], 0.0)
    vn = jnp.maximum(
        jnp.dot(h, w1_ref[...], preferred_element_type=jnp.float32)
        + b1_ref[...], 0.0)
    vo = vn + v
    if proj:
        wd_ref, ws_ref, vo_ref, pd_ref, ps_ref = rest
        vo_ref[...] = vo
        pd_ref[...] = jnp.dot(vo, wd_ref[...],
                              preferred_element_type=jnp.float32)
        ps_ref[...] = jnp.dot(vo, ws_ref[...],
                              preferred_element_type=jnp.float32)
    else:
        rest[0][...] = vo


def _node_mlp(parts_e, parts_c, v, w0a, w0v, b0, w1, b1, proj=None, blk=1000):
    in_specs = [
        pl.BlockSpec((NC, blk, H), lambda i: (0, i, 0)),
        pl.BlockSpec((blk, 1), lambda i: (i, 0)),
        pl.BlockSpec((blk, H), lambda i: (i, 0)),
        pl.BlockSpec((H, H), lambda i: (0, 0)),
        pl.BlockSpec((H, H), lambda i: (0, 0)),
        pl.BlockSpec((1, H), lambda i: (0, 0)),
        pl.BlockSpec((H, H), lambda i: (0, 0)),
        pl.BlockSpec((1, H), lambda i: (0, 0)),
    ]
    args = [parts_e, parts_c, v, w0a, w0v, b0.reshape(1, H), w1,
            b1.reshape(1, H)]
    n_out = 1
    if proj is not None:
        in_specs += [pl.BlockSpec((H, H), lambda i: (0, 0))] * 2
        args += [proj[0], proj[1]]
        n_out = 3
    out = pl.pallas_call(
        functools.partial(_node_body, proj is not None),
        grid=(N // blk,),
        in_specs=in_specs,
        out_specs=[pl.BlockSpec((blk, H), lambda i: (i, 0))] * n_out,
        out_shape=[jax.ShapeDtypeStruct((N, H), jnp.float32)] * n_out,
    )(*args)
    return out if proj is not None else out[0]


def _sc_gather_add(pd, ps, dstv, srcv):
    mesh = plsc.VectorSubcoreMesh(core_axis_name="c", subcore_axis_name="s")

    @functools.partial(
        pl.kernel,
        mesh=mesh,
        out_type=jax.ShapeDtypeStruct((E, H), jnp.float32),
        scratch_types=[
            pltpu.VMEM((C,), jnp.int32),
            pltpu.VMEM((C,), jnp.int32),
            pltpu.VMEM((C,), jnp.int32),
            pltpu.VMEM((C,), jnp.int32),
            pltpu.VMEM((C, H), jnp.float32),
            pltpu.VMEM((C, H), jnp.float32),
            pltpu.VMEM((C, H), jnp.float32),
            pltpu.VMEM((C, H), jnp.float32),
            pltpu.SemaphoreType.DMA,
            pltpu.SemaphoreType.DMA,
            pltpu.SemaphoreType.DMA,
            pltpu.SemaphoreType.DMA,
            pltpu.SemaphoreType.DMA,
            pltpu.SemaphoreType.DMA,
            pltpu.SemaphoreType.DMA,
            pltpu.SemaphoreType.DMA,
        ],
    )
    def k(pd_hbm, ps_hbm, dst_hbm, src_hbm, g_hbm,
          idxd0, idxs0, idxd1, idxs1, rd0, rs0, rd1, rs1,
          semd0, sems0, semd1, sems1, semi0, semi1, semo0, semo1):
        wid = lax.axis_index("s") * NC + lax.axis_index("c")
        base0 = wid * EPW
        bufs = ((idxd0, idxs0, rd0, rs0, semd0, sems0, semi0, semo0),
                (idxd1, idxs1, rd1, rs1, semd1, sems1, semi1, semo1))

        def idx_start(i, b):
            idxd, idxs = bufs[b][0], bufs[b][1]
            semi = bufs[b][6]
            base = base0 + i * C
            pltpu.async_copy(dst_hbm.at[pl.ds(base, C)], idxd, semi)
            pltpu.async_copy(src_hbm.at[pl.ds(base, C)], idxs, semi)

        def gather_start(b, first):
            idxd, idxs, rd, rs, semd, sems_, semi, semo = bufs[b]
            pltpu.make_async_copy(dst_hbm.at[pl.ds(0, C)], idxd, semi).wait()
            pltpu.make_async_copy(src_hbm.at[pl.ds(0, C)], idxs, semi).wait()
            if not first:
                pltpu.make_async_copy(rd, g_hbm.at[pl.ds(0, C)], semo).wait()
            pltpu.async_copy(pd_hbm.at[idxd], rd, semd)
            pltpu.async_copy(ps_hbm.at[idxs], rs, sems_)

        def finish(i, b):
            idxd, idxs, rd, rs, semd, sems_, _, semo = bufs[b]
            pltpu.make_async_copy(pd_hbm.at[idxd], rd, semd).wait()
            pltpu.make_async_copy(ps_hbm.at[idxs], rs, sems_).wait()

            def row(j, c2):
                for kk in range(8):
                    sl = pl.ds(kk * 16, 16)
                    rd[j, sl] = rd[j, sl] + rs[j, sl]
                return c2

            lax.fori_loop(0, C, row, 0)
            pltpu.async_copy(rd, g_hbm.at[pl.ds(base0 + i * C, C)], semo)

        idx_start(0, 0)
        gather_start(0, True)
        idx_start(1, 1)
        gather_start(1, True)

        def pair(p, carry):
            finish(2 * p, 0)
            idx_start(2 * p + 2, 0)
            gather_start(0, False)
            finish(2 * p + 1, 1)

            @pl.when(2 * p + 3 < NCHUNK)
            def _():
                idx_start(2 * p + 3, 1)
                gather_start(1, False)

            return carry

        lax.fori_loop(0, (NCHUNK - 1) // 2, pair, 0)
        finish(NCHUNK - 1, 0)
        pltpu.make_async_copy(rd0, g_hbm.at[pl.ds(0, C)], semo0).wait()
        pltpu.make_async_copy(rd1, g_hbm.at[pl.ds(0, C)], semo1).wait()

    return k(pd, ps, dstv, srcv)


def _sc_scatter_add(e_new, dstv):
    mesh = plsc.VectorSubcoreMesh(core_axis_name="c", subcore_axis_name="s")

    @functools.partial(
        pl.kernel,
        mesh=mesh,
        out_type=jax.ShapeDtypeStruct((NC, NPAD, H), jnp.float32),
        scratch_types=[
            pltpu.VMEM((C,), jnp.int32),
            pltpu.VMEM((C, H), jnp.float32),
            pltpu.VMEM((C,), jnp.int32),
            pltpu.VMEM((C, H), jnp.float32),
            pltpu.VMEM((160, H), jnp.float32),
            pltpu.VMEM_SHARED((NPAD, H), jnp.float32),
            pltpu.SemaphoreType.DMA,
            pltpu.SemaphoreType.DMA,
            pltpu.SemaphoreType.DMA,
            pltpu.SemaphoreType.DMA,
        ],
    )
    def k(e_hbm, dst_hbm, pe_hbm, idx0, ebuf0, idx1, ebuf1, zbe, she,
          semi0, seme0, semi1, seme1):
        cid = lax.axis_index("c")
        sid = lax.axis_index("s")
        wid = sid * NC + cid
        bufs = ((idx0, ebuf0, semi0, seme0), (idx1, ebuf1, semi1, seme1))

        def z1(j, c):
            for kk in range(8):
                zbe[j, pl.ds(kk * 16, 16)] = jnp.zeros((16,), jnp.float32)
            return c

        lax.fori_loop(0, 160, z1, 0)

        for t in range(NPT // 160):
            pltpu.sync_copy(zbe, she.at[pl.ds(sid * NPT + t * 160, 160)])
        plsc.subcore_barrier()

        base0 = wid * EPW

        def load_start(i, b):
            idx, ebuf, semi, seme = bufs[b]
            base = base0 + i * C
            pltpu.async_copy(dst_hbm.at[pl.ds(base, C)], idx, semi)
            pltpu.async_copy(e_hbm.at[pl.ds(base, C)], ebuf, seme)

        def do_scatter(b):
            idx, ebuf, semi, seme = bufs[b]
            pltpu.make_async_copy(dst_hbm.at[pl.ds(0, C)], idx, semi).wait()
            pltpu.make_async_copy(e_hbm.at[pl.ds(0, C)], ebuf, seme).wait()
            pltpu.sync_copy(ebuf, she.at[idx], add=True)

        load_start(0, 0)
        load_start(1, 1)

        def pair(p, carry):
            do_scatter(0)
            load_start(2 * p + 2, 0)
            do_scatter(1)

            @pl.when(2 * p + 3 < NCHUNK)
            def _():
                load_start(2 * p + 3, 1)

            return carry

        lax.fori_loop(0, (NCHUNK - 1) // 2, pair, 0)
        do_scatter(0)
        plsc.subcore_barrier()

        sl = pl.ds(sid * NPT, NPT)
        pltpu.sync_copy(she.at[sl], pe_hbm.at[cid, sl])

    return k(e_new, dstv)


def _sc_count(dstv):
    mesh = plsc.VectorSubcoreMesh(core_axis_name="c", subcore_axis_name="s")

    @functools.partial(
        pl.kernel,
        mesh=mesh,
        out_type=jax.ShapeDtypeStruct((NC, NPAD, H), jnp.float32),
        scratch_types=[
            pltpu.VMEM((C,), jnp.int32),
            pltpu.VMEM((C,), jnp.int32),
            pltpu.VMEM((C, H), jnp.float32),
            pltpu.VMEM((160, H), jnp.float32),
            pltpu.VMEM_SHARED((NPAD, H), jnp.float32),
            pltpu.SemaphoreType.DMA,
            pltpu.SemaphoreType.DMA,
        ],
    )
    def k(dst_hbm, pc_hbm, idx0, idx1, ones, zbe, shc, semi0, semi1):
        cid = lax.axis_index("c")
        sid = lax.axis_index("s")
        wid = sid * NC + cid

        def z1(j, c):
            for kk in range(8):
                zbe[j, pl.ds(kk * 16, 16)] = jnp.zeros((16,), jnp.float32)
            return c

        lax.fori_loop(0, 160, z1, 0)

        def o1(j, c):
            for kk in range(8):
                ones[j, pl.ds(kk * 16, 16)] = jnp.ones((16,), jnp.float32)
            return c

        lax.fori_loop(0, C, o1, 0)

        for t in range(NPT // 160):
            pltpu.sync_copy(zbe, shc.at[pl.ds(sid * NPT + t * 160, 160)])
        plsc.subcore_barrier()

        base0 = wid * EPW
        bufs = ((idx0, semi0), (idx1, semi1))

        def load_start(i, b):
            idx, semi = bufs[b]
            pltpu.async_copy(dst_hbm.at[pl.ds(base0 + i * C, C)], idx, semi)

        def do_scatter(b):
            idx, semi = bufs[b]
            pltpu.make_async_copy(dst_hbm.at[pl.ds(0, C)], idx, semi).wait()
            pltpu.sync_copy(ones, shc.at[idx], add=True)

        load_start(0, 0)
        load_start(1, 1)

        def pair(p, carry):
            do_scatter(0)
            load_start(2 * p + 2, 0)
            do_scatter(1)

            @pl.when(2 * p + 3 < NCHUNK)
            def _():
                load_start(2 * p + 3, 1)

            return carry

        lax.fori_loop(0, (NCHUNK - 1) // 2, pair, 0)
        do_scatter(0)
        plsc.subcore_barrier()

        sl = pl.ds(sid * NPT, NPT)
        pltpu.sync_copy(shc.at[sl], pc_hbm.at[cid, sl])

    return k(dstv)


def kernel(node_attr, edge_attr, edge_index, params):
    src = edge_index[0]
    dst = edge_index[1]

    def edge_w(i):
        w0 = params['edge_mlps'][i][0]
        return w0[0:H], w0[H:2 * H], w0[2 * H:3 * H]

    w0e, w0d, w0s = edge_w(0)
    v, pd, ps = _mlp2(node_attr, *params['enc_node'], True, 1000,
                      proj=(w0d, w0s))
    e = _mlp2(edge_attr, *params['enc_edge'], True, 2000,
              out_dtype=jnp.bfloat16)
    inv_c = _inv_cnt(_sc_count(dst))

    for i in range(STEPS):
        _, b0, w1, b1 = params['edge_mlps'][i]
        g = _sc_gather_add(pd, ps, dst, src)
        e_new, e_next = _edge_mlp(e, g, w0e, b0, w1, b1, i < STEPS - 1)
        parts_e = _sc_scatter_add(e_new, dst)
        nw0, nb0, nw1, nb1 = params['node_mlps'][i]
        if i < STEPS - 1:
            w0e, w0d, w0s = edge_w(i + 1)
            v, pd, ps = _node_mlp(parts_e, inv_c, v, nw0[0:H], nw0[H:2 * H],
                                  nb0, nw1, nb1, proj=(w0d, w0s))
        else:
            v = _node_mlp(parts_e, inv_c, v, nw0[0:H], nw0[H:2 * H], nb0,
                          nw1, nb1)
        e = e_next

    return _mlp2(v, *params['dec'], False, 1000)
